# Initial kernel scaffold; baseline (speedup 1.0000x reference)
import jax
import jax.numpy as jnp
from jax.experimental import pallas as pl


def kernel(x):
    raise NotImplementedError("write your pallas kernel here")


# speedup vs baseline: 6.4399x; 6.4399x over previous
import functools

import numpy as np
import jax
import jax.numpy as jnp
from jax import lax
from jax.experimental import pallas as pl
from jax.experimental.pallas import tpu as pltpu

TOPK = 5
MAX_CANDIDATES = 32
MAX_LAG_CAP = 168

PADB = 256
T = 1024
T1 = 1024


def _mean_kernel(x_ref, out_ref):
    out_ref[0, 0, :] = jnp.sum(x_ref[0], axis=-1) * (1.0 / x_ref.shape[2])


def _score_topk_kernel(lags_arr, m_ref, w_ref):
    B, L = m_ref.shape
    num = lags_arr.shape[0]
    m = m_ref[...]
    cols = []
    for j in range(num):
        lag = int(lags_arr[j])
        prod = m[:, : L - lag] * m[:, lag:]
        cols.append(jnp.sum(prod, axis=1, keepdims=True) * (1.0 / (L - lag)))
    scores = jnp.concatenate(cols, axis=1)

    iota = lax.broadcasted_iota(jnp.int32, (B, num), 1)
    neg_big = jnp.float32(-3.0e38)
    vals = []
    sel_idx = []
    s = scores
    for _ in range(TOPK):
        mx = jnp.max(s, axis=1, keepdims=True)
        idx = jnp.min(jnp.where(s == mx, iota, num), axis=1, keepdims=True)
        sel = iota == idx
        vals.append(mx)
        sel_idx.append(idx)
        s = jnp.where(sel, neg_big, s)
    vals = jnp.concatenate(vals, axis=1)
    idxs = jnp.concatenate(sel_idx, axis=1)
    denom = jnp.sum(vals, axis=1, keepdims=True) + 1e-6
    w = vals / denom
    w_full = jnp.zeros((B, num), jnp.float32)
    for k in range(TOPK):
        w_full = w_full + jnp.where(iota == idxs[:, k : k + 1],
                                    w[:, k : k + 1], 0.0)
    w_ref[...] = w_full


def _agg_kernel(lags_arr, w_ref, prev_ref, cur_ref, out_ref, scratch_ref):
    b = pl.program_id(0)
    scratch_ref[0:PADB, :] = prev_ref[0]
    scratch_ref[PADB:, :] = cur_ref[0]
    out_ref[0] = jnp.zeros_like(out_ref)[0]
    for j in range(lags_arr.shape[0]):
        lag = int(lags_arr[j])
        wj = w_ref[b, j]

        @pl.when(wj != 0.0)
        def _(lag=lag, wj=wj):
            out_ref[0] += wj * scratch_ref[PADB - lag : PADB - lag + T, :]


@jax.jit
def kernel(x):
    B, L, D = x.shape
    assert L % T == 0 and L % PADB == 0 and T % PADB == 0

    max_lag = min(L - 1, MAX_LAG_CAP)
    num = min(max_lag, MAX_CANDIDATES)
    lags_np = np.linspace(1.0, float(max_lag), num=num).astype(np.int64)

    x_mean = pl.pallas_call(
        _mean_kernel,
        grid=(B, L // T1),
        in_specs=[pl.BlockSpec((1, T1, D), lambda b, t: (b, t, 0))],
        out_specs=pl.BlockSpec((1, 1, T1), lambda b, t: (b, 0, t)),
        out_shape=jax.ShapeDtypeStruct((B, 1, L), jnp.float32),
    )(x)
    x_mean = x_mean.reshape(B, L)

    w32 = pl.pallas_call(
        functools.partial(_score_topk_kernel, lags_np),
        in_specs=[pl.BlockSpec(memory_space=pltpu.VMEM)],
        out_specs=pl.BlockSpec(memory_space=pltpu.VMEM),
        out_shape=jax.ShapeDtypeStruct((B, num), jnp.float32),
    )(x_mean)

    NPB = L // PADB
    R = T // PADB
    out = pl.pallas_call(
        functools.partial(_agg_kernel, lags_np),
        grid=(B, L // T),
        in_specs=[
            pl.BlockSpec(memory_space=pltpu.SMEM),
            pl.BlockSpec((1, PADB, D), lambda b, t: (b, (t * R - 1) % NPB, 0)),
            pl.BlockSpec((1, T, D), lambda b, t: (b, t, 0)),
        ],
        out_specs=pl.BlockSpec((1, T, D), lambda b, t: (b, t, 0)),
        out_shape=jax.ShapeDtypeStruct((B, L, D), jnp.float32),
        scratch_shapes=[pltpu.VMEM((PADB + T, D), jnp.float32)],
    )(w32, x, x)
    return out
